# baseline (speedup 1.0000x reference)
import functools

import jax
import jax.numpy as jnp
from jax import lax
from jax.experimental import pallas as pl
from jax.experimental.pallas import tpu as pltpu


def _gib_kernel(emb_ref, x_ref, adj_ref, brow_ref,
                c1w_ref, c1b_ref, c2wp_ref, c2b_ref,
                f1w_ref, f1b_ref, f2w_ref, f2b_ref,
                out_ref, pos_ref, gra_ref, assign_ref, pen_ref,
                c1wbf_ref, hacc_ref,
                *, nc, s, n, gk, kc, inv_nodes, inv_groups):
    i = pl.program_id(0)
    k = pl.program_id(1)
    p = x_ref.shape[-1]
    r = nc * s

    for kk in range(gk):
        @pl.when((i == 0) & (k == kk))
        def _(kk=kk):
            c1wbf_ref[kk * kc:(kk + 1) * kc, :] = (
                c1w_ref[...].astype(jnp.bfloat16))

    ebf = emb_ref[...].astype(jnp.bfloat16)
    for kk in range(gk):
        @pl.when(k == kk)
        def _(kk=kk):
            part = jnp.dot(ebf, c1wbf_ref[kk * kc:(kk + 1) * kc, :],
                           preferred_element_type=jnp.float32)
            if kk == 0:
                hacc_ref[...] = part + c1b_ref[...]
            else:
                hacc_ref[...] += part

    @pl.when(k == gk - 1)
    def _():
        h = jnp.maximum(hacc_ref[...], 0.0)

        nb = p // 128
        cwt = jnp.transpose(c2wp_ref[...])
        db = c2b_ref[:, 1:2] - c2b_ref[:, 0:1]
        diff = None
        for j in range(nb):
            dwj = cwt[:, 2 * j + 1:2 * j + 2] - cwt[:, 2 * j:2 * j + 1]
            term = jnp.dot(h[:, 128 * j:128 * (j + 1)], dwj,
                           preferred_element_type=jnp.float32)
            diff = term if diff is None else diff + term
        diff = diff + db

        a0 = 1.0 / (1.0 + jnp.exp(diff))
        b0 = a0.reshape(nc, s)
        assign_ref[...] = jnp.concatenate(
            [b0.reshape(nc, 1, s), (1.0 - b0).reshape(nc, 1, s)],
            axis=1).reshape(2 * nc, s)

        x = x_ref[...]
        pos = jnp.sum((a0 * x).reshape(nc, s, p), axis=1) * inv_nodes
        gra = jnp.sum(x.reshape(nc, s, p), axis=1) * inv_nodes
        pos_ref[...] = pos
        gra_ref[...] = gra

        adj = adj_ref[...]
        t0 = jnp.dot(b0, adj, preferred_element_type=jnp.float32)
        t1 = jnp.sum(adj, axis=0, keepdims=True) - t0
        e00 = jnp.sum(t0 * b0, axis=1, keepdims=True)
        e01 = jnp.sum(t0, axis=1, keepdims=True) - e00
        e10 = jnp.sum(t1 * b0, axis=1, keepdims=True)
        e11 = jnp.sum(t1, axis=1, keepdims=True) - e10
        n0 = jnp.maximum(jnp.abs(e00) + jnp.abs(e01), 1e-5)
        n1 = jnp.maximum(jnp.abs(e10) + jnp.abs(e11), 1e-5)
        d0 = e00 / n0
        d1 = e11 / n1
        mse = 0.5 * ((d0 - 1.0) ** 2 + (d1 - 1.0) ** 2)

        brow = brow_ref[...]
        bcol = brow.reshape(n, 1)
        eq = (bcol == brow).astype(jnp.float32)
        counts = jnp.sum(eq, axis=1, keepdims=True)
        rowi = lax.broadcasted_iota(jnp.int32, (nc, n), 0)
        coli = lax.broadcasted_iota(jnp.int32, (nc, n), 1)
        sel = (coli == rowi + i * nc).astype(jnp.float32)
        wchunk = jnp.dot(sel, inv_groups / counts,
                         preferred_element_type=jnp.float32)
        part = jnp.sum(wchunk * mse).reshape(1, 1)

        @pl.when(i == 0)
        def _():
            pen_ref[...] = part

        @pl.when(i > 0)
        def _():
            pen_ref[...] += part

        hh = jnp.dot(pos, f1w_ref[...],
                     preferred_element_type=jnp.float32) + f1b_ref[...]
        hh = jnp.maximum(hh, 0.0)
        out_ref[...] = jnp.dot(hh, f2w_ref[...],
                               preferred_element_type=jnp.float32) + f2b_ref[...]


def kernel(emb, adj, batch, prot_feature,
           c1_w, c1_b, c2_w, c2_b, fc1_w, fc1_b, fc2_w, fc2_b):
    N, S, DIN = emb.shape
    P = prot_feature.shape[-1]
    H = fc2_w.shape[-1]
    NS = N * S
    G = 2
    GK = 5
    KC = DIN // GK
    NC = N // G
    R = NS // G
    NUM_GROUPS = 4

    emb2d = emb.reshape(NS, DIN)
    x2d = prot_feature.reshape(NS, P)
    brow = batch.astype(jnp.int32).reshape(1, N)
    NB = P // 128
    c2wp = c2_w.T.reshape(2, NB, 128).transpose(1, 0, 2).reshape(2 * NB, 128)

    body = functools.partial(_gib_kernel, nc=NC, s=S, n=N, gk=GK, kc=KC,
                             inv_nodes=1.0 / S, inv_groups=1.0 / NUM_GROUPS)

    out_shapes = (
        jax.ShapeDtypeStruct((N, H), jnp.float32),
        jax.ShapeDtypeStruct((N, P), jnp.float32),
        jax.ShapeDtypeStruct((N, P), jnp.float32),
        jax.ShapeDtypeStruct((2 * N, S), jnp.float32),
        jax.ShapeDtypeStruct((1, 1), jnp.float32),
    )

    out, pos, gra, assign, pen = pl.pallas_call(
        body,
        out_shape=out_shapes,
        grid=(G, GK),
        in_specs=[
            pl.BlockSpec((R, KC), lambda i, k: (i, k)),
            pl.BlockSpec((R, P), lambda i, k: (i, 0)),
            pl.BlockSpec((S, S), lambda i, k: (0, 0)),
            pl.BlockSpec((1, N), lambda i, k: (0, 0)),
            pl.BlockSpec((KC, P),
                         lambda i, k: (jnp.where(i == 0, k, GK - 1), 0)),
            pl.BlockSpec((1, P), lambda i, k: (0, 0)),
            pl.BlockSpec((2 * NB, 128), lambda i, k: (0, 0)),
            pl.BlockSpec((1, 2), lambda i, k: (0, 0)),
            pl.BlockSpec((P, P), lambda i, k: (0, 0)),
            pl.BlockSpec((1, P), lambda i, k: (0, 0)),
            pl.BlockSpec((P, H), lambda i, k: (0, 0)),
            pl.BlockSpec((1, H), lambda i, k: (0, 0)),
        ],
        out_specs=[
            pl.BlockSpec((NC, H), lambda i, k: (i, 0)),
            pl.BlockSpec((NC, P), lambda i, k: (i, 0)),
            pl.BlockSpec((NC, P), lambda i, k: (i, 0)),
            pl.BlockSpec((2 * NC, S), lambda i, k: (i, 0)),
            pl.BlockSpec((1, 1), lambda i, k: (0, 0)),
        ],
        scratch_shapes=[pltpu.VMEM((DIN, P), jnp.bfloat16),
                        pltpu.VMEM((R, P), jnp.float32)],
        compiler_params=pltpu.CompilerParams(
            dimension_semantics=("arbitrary", "arbitrary")),
    )(emb2d, x2d, adj, brow,
      c1_w, c1_b, c2wp, c2_b, fc1_w, fc1_b, fc2_w, fc2_b)

    assignment = assign.reshape(N, 2, S).transpose(0, 2, 1)
    return out, pos, gra, pen[0, 0], assignment


# speedup vs baseline: 1.4559x; 1.4559x over previous
import functools

import jax
import jax.numpy as jnp
from jax import lax
from jax.experimental import pallas as pl
from jax.experimental.pallas import tpu as pltpu


def _gib_kernel(emb_ref, x_ref, adj_ref, brow_ref,
                c1w_ref, c1b_ref, c2wp_ref, c2b_ref,
                f1w_ref, f1b_ref, f2w_ref, f2b_ref,
                out_ref, pos_ref, gra_ref, assign_ref, pen_ref,
                c1wbf_ref, xv_ref, f1v_ref, f2v_ref,
                semx_ref, semf_ref,
                *, nc, s, n, inv_nodes, inv_groups):
    i = pl.program_id(0)
    p = xv_ref.shape[-1]
    r = nc * s

    cpx = pltpu.make_async_copy(x_ref.at[pl.ds(i * r, r), :], xv_ref, semx_ref)
    cpx.start()
    cpf1 = pltpu.make_async_copy(f1w_ref, f1v_ref, semf_ref.at[0])
    cpf2 = pltpu.make_async_copy(f2w_ref, f2v_ref, semf_ref.at[1])

    @pl.when(i == 0)
    def _():
        cpf1.start()
        cpf2.start()
        c1wbf_ref[...] = c1w_ref[...].astype(jnp.bfloat16)

    e = emb_ref[...]
    h = jnp.dot(e.astype(jnp.bfloat16), c1wbf_ref[...],
                preferred_element_type=jnp.float32) + c1b_ref[...]
    h = jnp.maximum(h, 0.0)
    nb = p // 128
    cwt = jnp.transpose(c2wp_ref[...])
    db = c2b_ref[:, 1:2] - c2b_ref[:, 0:1]
    diff = None
    for k in range(nb):
        dwk = cwt[:, 2 * k + 1:2 * k + 2] - cwt[:, 2 * k:2 * k + 1]
        term = jnp.dot(h[:, 128 * k:128 * (k + 1)], dwk,
                       preferred_element_type=jnp.float32)
        diff = term if diff is None else diff + term
    diff = diff + db

    a0 = 1.0 / (1.0 + jnp.exp(diff))
    b0 = a0.reshape(nc, s)
    assign_ref[...] = jnp.concatenate(
        [b0.reshape(nc, 1, s), (1.0 - b0).reshape(nc, 1, s)],
        axis=1).reshape(2 * nc, s)

    cpx.wait()
    x = xv_ref[...]
    pos = jnp.sum((a0 * x).reshape(nc, s, p), axis=1) * inv_nodes
    gra = jnp.sum(x.reshape(nc, s, p), axis=1) * inv_nodes
    pos_ref[...] = pos
    gra_ref[...] = gra

    adj = adj_ref[...]
    t0 = jnp.dot(b0, adj, preferred_element_type=jnp.float32)
    t1 = jnp.sum(adj, axis=0, keepdims=True) - t0
    e00 = jnp.sum(t0 * b0, axis=1, keepdims=True)
    e01 = jnp.sum(t0, axis=1, keepdims=True) - e00
    e10 = jnp.sum(t1 * b0, axis=1, keepdims=True)
    e11 = jnp.sum(t1, axis=1, keepdims=True) - e10
    n0 = jnp.maximum(jnp.abs(e00) + jnp.abs(e01), 1e-5)
    n1 = jnp.maximum(jnp.abs(e10) + jnp.abs(e11), 1e-5)
    d0 = e00 / n0
    d1 = e11 / n1
    mse = 0.5 * ((d0 - 1.0) ** 2 + (d1 - 1.0) ** 2)

    brow = brow_ref[...]
    bcol = brow.reshape(n, 1)
    eq = (bcol == brow).astype(jnp.float32)
    counts = jnp.sum(eq, axis=1, keepdims=True)
    rowi = lax.broadcasted_iota(jnp.int32, (nc, n), 0)
    coli = lax.broadcasted_iota(jnp.int32, (nc, n), 1)
    sel = (coli == rowi + i * nc).astype(jnp.float32)
    wchunk = jnp.dot(sel, inv_groups / counts,
                     preferred_element_type=jnp.float32)
    part = jnp.sum(wchunk * mse).reshape(1, 1)

    @pl.when(i == 0)
    def _():
        pen_ref[...] = part

    @pl.when(i > 0)
    def _():
        pen_ref[...] += part

    @pl.when(i == 0)
    def _():
        cpf1.wait()
        cpf2.wait()

    hh = jnp.dot(pos, f1v_ref[...],
                 preferred_element_type=jnp.float32) + f1b_ref[...]
    hh = jnp.maximum(hh, 0.0)
    out_ref[...] = jnp.dot(hh, f2v_ref[...],
                           preferred_element_type=jnp.float32) + f2b_ref[...]


def kernel(emb, adj, batch, prot_feature,
           c1_w, c1_b, c2_w, c2_b, fc1_w, fc1_b, fc2_w, fc2_b):
    N, S, DIN = emb.shape
    P = prot_feature.shape[-1]
    H = fc2_w.shape[-1]
    NS = N * S
    G = 2
    NC = N // G
    R = NS // G
    NUM_GROUPS = 4

    emb2d = emb.reshape(NS, DIN)
    x2d = prot_feature.reshape(NS, P)
    brow = batch.astype(jnp.int32).reshape(1, N)
    NB = P // 128
    c2wp = c2_w.T.reshape(2, NB, 128).transpose(1, 0, 2).reshape(2 * NB, 128)

    body = functools.partial(_gib_kernel, nc=NC, s=S, n=N,
                             inv_nodes=1.0 / S, inv_groups=1.0 / NUM_GROUPS)

    out_shapes = (
        jax.ShapeDtypeStruct((N, H), jnp.float32),
        jax.ShapeDtypeStruct((N, P), jnp.float32),
        jax.ShapeDtypeStruct((N, P), jnp.float32),
        jax.ShapeDtypeStruct((2 * N, S), jnp.float32),
        jax.ShapeDtypeStruct((1, 1), jnp.float32),
    )

    out, pos, gra, assign, pen = pl.pallas_call(
        body,
        out_shape=out_shapes,
        grid=(G,),
        in_specs=[
            pl.BlockSpec((R, DIN), lambda i: (i, 0)),
            pl.BlockSpec(memory_space=pl.ANY),
            pl.BlockSpec((S, S), lambda i: (0, 0)),
            pl.BlockSpec((1, N), lambda i: (0, 0)),
            pl.BlockSpec((DIN, P), lambda i: (0, 0)),
            pl.BlockSpec((1, P), lambda i: (0, 0)),
            pl.BlockSpec((2 * NB, 128), lambda i: (0, 0)),
            pl.BlockSpec((1, 2), lambda i: (0, 0)),
            pl.BlockSpec(memory_space=pl.ANY),
            pl.BlockSpec((1, P), lambda i: (0, 0)),
            pl.BlockSpec(memory_space=pl.ANY),
            pl.BlockSpec((1, H), lambda i: (0, 0)),
        ],
        out_specs=[
            pl.BlockSpec((NC, H), lambda i: (i, 0)),
            pl.BlockSpec((NC, P), lambda i: (i, 0)),
            pl.BlockSpec((NC, P), lambda i: (i, 0)),
            pl.BlockSpec((2 * NC, S), lambda i: (i, 0)),
            pl.BlockSpec((1, 1), lambda i: (0, 0)),
        ],
        scratch_shapes=[pltpu.VMEM((DIN, P), jnp.bfloat16),
                        pltpu.VMEM((R, P), jnp.float32),
                        pltpu.VMEM((P, P), jnp.float32),
                        pltpu.VMEM((P, H), jnp.float32),
                        pltpu.SemaphoreType.DMA,
                        pltpu.SemaphoreType.DMA((2,))],
        compiler_params=pltpu.CompilerParams(
            dimension_semantics=("arbitrary",)),
    )(emb2d, x2d, adj, brow,
      c1_w, c1_b, c2wp, c2_b, fc1_w, fc1_b, fc2_w, fc2_b)

    assignment = assign.reshape(N, 2, S).transpose(0, 2, 1)
    return out, pos, gra, pen[0, 0], assignment
